# baseline (speedup 1.0000x reference)
import functools

import jax
import jax.numpy as jnp
from jax import lax
from jax.experimental import pallas as pl
from jax.experimental.pallas import tpu as pltpu
from jax.experimental.pallas import tpu_sc as plsc

_NUM_EMB = 1000000
_EMB_DIM = 64
_BATCH = 16384
_HIST = 20
_QMAX_F = 65535.0
_EPS = 0.0001 / 65535
_MAGIC = 12582912.0

_NW = 32
_C = 512
_IDX_MINOR = 128
_C_ROWS = _C // _IDX_MINOR
_SUPER = 1024
_BCHUNKS = _BATCH // _SUPER
_NTILE = _HIST * _BCHUNKS
_TPW = _NTILE // _NW

_SPLIT = 512000
_BW = 4096
_GRID_A = _SPLIT // _BW


def _qparams_body(wa_ref, wb_ref, out_ref, scale_ref, zp_ref, acc_ref):
    i = pl.program_id(0)
    a = wa_ref[...]
    b = wb_ref[...]
    colb = (
        _SPLIT + i * _BW
        + lax.broadcasted_iota(jnp.int32, (_EMB_DIM, _BW), 1)
    )
    validb = colb < _NUM_EMB
    out_ref[:, 0:_EMB_DIM] = a.T
    out_ref[:, _EMB_DIM:2 * _EMB_DIM] = b.T
    bmin = jnp.minimum(jnp.min(a), jnp.min(jnp.where(validb, b, jnp.inf)))
    bmax = jnp.maximum(jnp.max(a), jnp.max(jnp.where(validb, b, -jnp.inf)))

    @pl.when(i == 0)
    def _init():
        acc_ref[0] = bmin
        acc_ref[1] = bmax

    @pl.when(i > 0)
    def _acc():
        acc_ref[0] = jnp.minimum(acc_ref[0], bmin)
        acc_ref[1] = jnp.maximum(acc_ref[1], bmax)

    @pl.when(i == pl.num_programs(0) - 1)
    def _finish():
        mn = jnp.minimum(acc_ref[0], 0.0)
        mx = jnp.maximum(acc_ref[1], 0.0)
        sc = jnp.maximum((mx - mn) / _QMAX_F, jnp.float32(_EPS))
        zp = jnp.clip(-jnp.round(mn / sc), 0.0, _QMAX_F)
        scale_ref[0, 0] = sc
        zp_ref[0, 0] = zp


def _tc_qparams(wt):
    w_rm, scale, zp = pl.pallas_call(
        _qparams_body,
        grid=(_GRID_A,),
        in_specs=[
            pl.BlockSpec((_EMB_DIM, _BW), lambda i: (0, i)),
            pl.BlockSpec(
                (_EMB_DIM, _BW),
                lambda i: (0, jnp.minimum(i + _GRID_A, _NUM_EMB // _BW)),
            ),
        ],
        out_specs=[
            pl.BlockSpec((_BW, 2 * _EMB_DIM), lambda i: (i, 0)),
            pl.BlockSpec(memory_space=pltpu.SMEM),
            pl.BlockSpec(memory_space=pltpu.SMEM),
        ],
        out_shape=[
            jax.ShapeDtypeStruct((_SPLIT, 2 * _EMB_DIM), jnp.float32),
            jax.ShapeDtypeStruct((1, 1), jnp.float32),
            jax.ShapeDtypeStruct((1, 1), jnp.float32),
        ],
        scratch_shapes=[pltpu.SMEM((2,), jnp.float32)],
    )(wt, wt)
    return w_rm, scale[0, 0], zp[0, 0]


def _sc_gather_quant(xp3, weight, params):
    mesh = plsc.VectorSubcoreMesh(core_axis_name="c", subcore_axis_name="s")

    @functools.partial(
        pl.kernel,
        mesh=mesh,
        compiler_params=pltpu.CompilerParams(
            use_tc_tiling_on_sc=False, needs_layout_passes=False
        ),
        out_type=jax.ShapeDtypeStruct((_HIST, _EMB_DIM, _BATCH), jnp.float32),
        scratch_types=[
            pltpu.VMEM((_SUPER // _IDX_MINOR, _IDX_MINOR), jnp.int32),
            pltpu.VMEM((_C, _EMB_DIM), jnp.float32),
            pltpu.VMEM((_EMB_DIM, _C + 1), jnp.float32),
            pltpu.VMEM((4, 16), jnp.float32),
            pltpu.SemaphoreType.DMA,
        ],
    )
    def k(x_hbm, w_hbm, p_hbm, out_hbm, idx_v, rows_v, out_t, p_v, sem):
        wid = lax.axis_index("s") * 2 + lax.axis_index("c")
        pltpu.sync_copy(p_hbm, p_v)
        inv_scale = p_v[0, :]
        zp = p_v[1, :]
        scale = p_v[2, :]

        def tile_body(k_i, carry):
            st = wid * _TPW + k_i
            h = st // _BCHUNKS
            sc_i = st % _BCHUNKS
            pltpu.sync_copy(
                x_hbm.at[h, pl.ds(sc_i * (_SUPER // _IDX_MINOR), _SUPER // _IDX_MINOR)],
                idx_v,
            )
            for half in range(_SUPER // _C):
                copies = [
                    pltpu.async_copy(
                        w_hbm.at[idx_v.at[half * _C_ROWS + j]],
                        rows_v.at[pl.ds(j * _IDX_MINOR, _IDX_MINOR)],
                        sem,
                    )
                    for j in range(_C_ROWS)
                ]
                for cp in copies:
                    cp.wait()

                def row_body(r, c2):
                    col = jnp.zeros((16,), jnp.int32) + r
                    for j in range(_EMB_DIM // 16):
                        v = rows_v[r, pl.ds(j * 16, 16)]
                        t = v * inv_scale + zp
                        t = jnp.minimum(jnp.maximum(t, 0.0), _QMAX_F)
                        t = (t + _MAGIC) - _MAGIC
                        plsc.store_scatter(
                            out_t,
                            [jnp.arange(16, dtype=jnp.int32) + (j * 16), col],
                            (t - zp) * scale,
                        )
                    return c2

                lax.fori_loop(0, _C, row_body, 0)
                base = pl.multiple_of(sc_i * _SUPER + half * _C, _C)
                pltpu.sync_copy(
                    out_t.at[:, pl.ds(0, _C)],
                    out_hbm.at[h, :, pl.ds(base, _C)],
                )
            return carry

        lax.fori_loop(0, _TPW, tile_body, 0)

    return k(xp3, weight, params)


def kernel(x, weight):
    w_rm, scale, zp = _tc_qparams(weight.T)
    w_lin = w_rm.reshape(2 * _SPLIT, _EMB_DIM)
    inv_scale = 1.0 / scale
    params = jnp.stack(
        [
            jnp.full((16,), inv_scale, jnp.float32),
            jnp.full((16,), zp, jnp.float32),
            jnp.full((16,), scale, jnp.float32),
            jnp.zeros((16,), jnp.float32),
        ]
    )
    xi = x.T.astype(jnp.int32)
    xm = jnp.where(xi < _SPLIT, 2 * xi, 2 * (xi - _SPLIT) + 1)
    xp3 = xm.reshape(_HIST, _BATCH // _IDX_MINOR, _IDX_MINOR)
    out3 = _sc_gather_quant(xp3, w_lin, params)
    return out3.transpose(2, 0, 1)


# speedup vs baseline: 1.4711x; 1.4711x over previous
import functools

import jax
import jax.numpy as jnp
from jax import lax
from jax.experimental import pallas as pl
from jax.experimental.pallas import tpu as pltpu
from jax.experimental.pallas import tpu_sc as plsc

_NUM_EMB = 1000000
_EMB_DIM = 64
_BATCH = 16384
_HIST = 20
_QMAX_F = 65535.0
_EPS = 0.0001 / 65535
_MAGIC = 12582912.0

_NW = 32
_C = 512
_IDX_MINOR = 128
_C_ROWS = _C // _IDX_MINOR
_SUPER = 1024
_BCHUNKS = _BATCH // _SUPER
_NTILE = _HIST * _BCHUNKS
_TPW = _NTILE // _NW

_SPLIT = 512000
_BW = 4096
_GRID_A = _SPLIT // _BW


def _qparams_body(wa_ref, wb_ref, out_ref, scale_ref, zp_ref, acc_ref):
    i = pl.program_id(0)
    a = wa_ref[...]
    b = wb_ref[...]
    colb = (
        _SPLIT + i * _BW
        + lax.broadcasted_iota(jnp.int32, (_EMB_DIM, _BW), 1)
    )
    validb = colb < _NUM_EMB
    out_ref[:, 0:_EMB_DIM] = a.T
    out_ref[:, _EMB_DIM:2 * _EMB_DIM] = b.T
    bmin = jnp.minimum(jnp.min(a), jnp.min(jnp.where(validb, b, jnp.inf)))
    bmax = jnp.maximum(jnp.max(a), jnp.max(jnp.where(validb, b, -jnp.inf)))

    @pl.when(i == 0)
    def _init():
        acc_ref[0] = bmin
        acc_ref[1] = bmax

    @pl.when(i > 0)
    def _acc():
        acc_ref[0] = jnp.minimum(acc_ref[0], bmin)
        acc_ref[1] = jnp.maximum(acc_ref[1], bmax)

    @pl.when(i == pl.num_programs(0) - 1)
    def _finish():
        mn = jnp.minimum(acc_ref[0], 0.0)
        mx = jnp.maximum(acc_ref[1], 0.0)
        sc = jnp.maximum((mx - mn) / _QMAX_F, jnp.float32(_EPS))
        zp = jnp.clip(-jnp.round(mn / sc), 0.0, _QMAX_F)
        scale_ref[0, 0] = sc
        zp_ref[0, 0] = zp


def _tc_qparams(wt):
    w_rm, scale, zp = pl.pallas_call(
        _qparams_body,
        grid=(_GRID_A,),
        in_specs=[
            pl.BlockSpec((_EMB_DIM, _BW), lambda i: (0, i)),
            pl.BlockSpec(
                (_EMB_DIM, _BW),
                lambda i: (0, jnp.minimum(i + _GRID_A, _NUM_EMB // _BW)),
            ),
        ],
        out_specs=[
            pl.BlockSpec((_BW, 2 * _EMB_DIM), lambda i: (i, 0)),
            pl.BlockSpec(memory_space=pltpu.SMEM),
            pl.BlockSpec(memory_space=pltpu.SMEM),
        ],
        out_shape=[
            jax.ShapeDtypeStruct((_SPLIT, 2 * _EMB_DIM), jnp.float32),
            jax.ShapeDtypeStruct((1, 1), jnp.float32),
            jax.ShapeDtypeStruct((1, 1), jnp.float32),
        ],
        scratch_shapes=[pltpu.SMEM((2,), jnp.float32)],
    )(wt, wt)
    return w_rm, scale[0, 0], zp[0, 0]


def _sc_gather_quant(xp3, weight, params):
    mesh = plsc.VectorSubcoreMesh(core_axis_name="c", subcore_axis_name="s")

    @functools.partial(
        pl.kernel,
        mesh=mesh,
        compiler_params=pltpu.CompilerParams(
            use_tc_tiling_on_sc=False, needs_layout_passes=False
        ),
        out_type=jax.ShapeDtypeStruct((_HIST, _BATCH, _EMB_DIM), jnp.float32),
        scratch_types=[
            pltpu.VMEM((2 * _SUPER // _IDX_MINOR, _IDX_MINOR), jnp.int32),
            pltpu.VMEM((_C, _EMB_DIM), jnp.float32),
            pltpu.VMEM((_C, _EMB_DIM), jnp.float32),
            pltpu.VMEM((4, 16), jnp.float32),
            pltpu.SemaphoreType.DMA,
            pltpu.SemaphoreType.DMA,
        ],
    )
    def k(x_hbm, w_hbm, p_hbm, out_hbm, idx_v, rows_a, rows_b, p_v, sem_a, sem_b):
        wid = lax.axis_index("s") * 2 + lax.axis_index("c")
        pltpu.sync_copy(p_hbm, p_v)
        inv_scale = p_v[0, :]
        zp = p_v[1, :]
        scale = p_v[2, :]
        nrows = _SUPER // _IDX_MINOR

        def hs(s):
            st = wid * _TPW + s
            return st // _BCHUNKS, st % _BCHUNKS

        def fetch_idx(s):
            h, sc_i = hs(s)
            pltpu.sync_copy(
                x_hbm.at[h, pl.ds(sc_i * nrows, nrows)],
                idx_v.at[pl.ds((s % 2) * nrows, nrows)],
            )

        def fire_gather(c):
            s, half = divmod(c, 2)
            rows, sem = (rows_a, sem_a) if c % 2 == 0 else (rows_b, sem_b)
            return [
                pltpu.async_copy(
                    w_hbm.at[idx_v.at[(s % 2) * nrows + half * _C_ROWS + j]],
                    rows.at[pl.ds(j * _IDX_MINOR, _IDX_MINOR)],
                    sem,
                )
                for j in range(_C_ROWS)
            ]

        nchunks = 2 * _TPW
        inflight = [None] * nchunks
        fetch_idx(0)
        inflight[0] = fire_gather(0)
        for c in range(nchunks):
            s, half = divmod(c, 2)
            rows = rows_a if c % 2 == 0 else rows_b
            if c + 1 < nchunks:
                if (c + 1) % 2 == 0:
                    fetch_idx(s + 1)
                inflight[c + 1] = fire_gather(c + 1)
            for cp in inflight[c]:
                cp.wait()

            def row_body(r, c2, rows=rows):
                for j in range(_EMB_DIM // 16):
                    v = rows[r, pl.ds(j * 16, 16)]
                    t = v * inv_scale + zp
                    t = jnp.minimum(jnp.maximum(t, 0.0), _QMAX_F)
                    t = (t + _MAGIC) - _MAGIC
                    rows[r, pl.ds(j * 16, 16)] = (t - zp) * scale
                return c2

            lax.fori_loop(0, _C, row_body, 0)
            h, sc_i = hs(s)
            base = pl.multiple_of(sc_i * _SUPER + half * _C, _C)
            pltpu.sync_copy(rows, out_hbm.at[h, pl.ds(base, _C)])

    return k(xp3, weight, params)


def kernel(x, weight):
    w_rm, scale, zp = _tc_qparams(weight.T)
    w_lin = w_rm.reshape(2 * _SPLIT, _EMB_DIM)
    inv_scale = 1.0 / scale
    params = jnp.stack(
        [
            jnp.full((16,), inv_scale, jnp.float32),
            jnp.full((16,), zp, jnp.float32),
            jnp.full((16,), scale, jnp.float32),
            jnp.zeros((16,), jnp.float32),
        ]
    )
    xi = x.T.astype(jnp.int32)
    xm = jnp.where(xi < _SPLIT, 2 * xi, 2 * (xi - _SPLIT) + 1)
    xp3 = xm.reshape(_HIST, _BATCH // _IDX_MINOR, _IDX_MINOR)
    out3 = _sc_gather_quant(xp3, w_lin, params)
    return out3.transpose(2, 0, 1)


# speedup vs baseline: 1.4729x; 1.0012x over previous
import functools

import jax
import jax.numpy as jnp
from jax import lax
from jax.experimental import pallas as pl
from jax.experimental.pallas import tpu as pltpu
from jax.experimental.pallas import tpu_sc as plsc

_NUM_EMB = 1000000
_EMB_DIM = 64
_BATCH = 16384
_HIST = 20
_QMAX_F = 65535.0
_EPS = 0.0001 / 65535
_MAGIC = 12582912.0

_NW = 32
_C = 512
_IDX_MINOR = 128
_C_ROWS = _C // _IDX_MINOR
_SUPER = 1024
_BCHUNKS = _BATCH // _SUPER
_NTILE = _HIST * _BCHUNKS
_TPW = _NTILE // _NW

_SPLIT = 512000
_BW = 4096
_GRID_A = _SPLIT // _BW


def _qparams_body(wa_ref, wb_ref, out_ref, scale_ref, zp_ref, acc_ref):
    i = pl.program_id(0)
    a = wa_ref[...]
    b = wb_ref[...]
    colb = (
        _SPLIT + i * _BW
        + lax.broadcasted_iota(jnp.int32, (_EMB_DIM, _BW), 1)
    )
    validb = colb < _NUM_EMB
    out_ref[:, 0:_EMB_DIM] = a.T
    out_ref[:, _EMB_DIM:2 * _EMB_DIM] = b.T
    bmin = jnp.minimum(jnp.min(a), jnp.min(jnp.where(validb, b, jnp.inf)))
    bmax = jnp.maximum(jnp.max(a), jnp.max(jnp.where(validb, b, -jnp.inf)))

    @pl.when(i == 0)
    def _init():
        acc_ref[0] = bmin
        acc_ref[1] = bmax

    @pl.when(i > 0)
    def _acc():
        acc_ref[0] = jnp.minimum(acc_ref[0], bmin)
        acc_ref[1] = jnp.maximum(acc_ref[1], bmax)

    @pl.when(i == pl.num_programs(0) - 1)
    def _finish():
        mn = jnp.minimum(acc_ref[0], 0.0)
        mx = jnp.maximum(acc_ref[1], 0.0)
        sc = jnp.maximum((mx - mn) / _QMAX_F, jnp.float32(_EPS))
        zp = jnp.clip(-jnp.round(mn / sc), 0.0, _QMAX_F)
        scale_ref[0, 0] = sc
        zp_ref[0, 0] = zp


def _tc_qparams(wt):
    w_rm, scale, zp = pl.pallas_call(
        _qparams_body,
        grid=(_GRID_A,),
        in_specs=[
            pl.BlockSpec((_EMB_DIM, _BW), lambda i: (0, i)),
            pl.BlockSpec(
                (_EMB_DIM, _BW),
                lambda i: (0, jnp.minimum(i + _GRID_A, _NUM_EMB // _BW)),
            ),
        ],
        out_specs=[
            pl.BlockSpec((_BW, 2 * _EMB_DIM), lambda i: (i, 0)),
            pl.BlockSpec(memory_space=pltpu.SMEM),
            pl.BlockSpec(memory_space=pltpu.SMEM),
        ],
        out_shape=[
            jax.ShapeDtypeStruct((_SPLIT, 2 * _EMB_DIM), jnp.float32),
            jax.ShapeDtypeStruct((1, 1), jnp.float32),
            jax.ShapeDtypeStruct((1, 1), jnp.float32),
        ],
        scratch_shapes=[pltpu.SMEM((2,), jnp.float32)],
    )(wt, wt)
    return w_rm, scale[0, 0], zp[0, 0]


def _sc_gather_quant(xp3, weight, params):
    mesh = plsc.VectorSubcoreMesh(core_axis_name="c", subcore_axis_name="s")

    @functools.partial(
        pl.kernel,
        mesh=mesh,
        compiler_params=pltpu.CompilerParams(
            use_tc_tiling_on_sc=False, needs_layout_passes=False
        ),
        out_type=jax.ShapeDtypeStruct((_HIST, _BATCH, _EMB_DIM), jnp.float32),
        scratch_types=[
            pltpu.VMEM((2 * _SUPER // _IDX_MINOR, _IDX_MINOR), jnp.int32),
            pltpu.VMEM((_C, _EMB_DIM), jnp.float32),
            pltpu.VMEM((_C, _EMB_DIM), jnp.float32),
            pltpu.VMEM((4, 16), jnp.float32),
            pltpu.SemaphoreType.DMA,
            pltpu.SemaphoreType.DMA,
        ],
    )
    def k(x_hbm, w_hbm, p_hbm, out_hbm, idx_v, rows_a, rows_b, p_v, sem_a, sem_b):
        wid = lax.axis_index("s") * 2 + lax.axis_index("c")
        pltpu.sync_copy(p_hbm, p_v)
        inv_scale = p_v[0, :]
        zp = p_v[1, :]
        scale = p_v[2, :]
        nrows = _SUPER // _IDX_MINOR

        def hs(s):
            st = wid * _TPW + s
            return st // _BCHUNKS, st % _BCHUNKS

        def fetch_idx(s):
            h, sc_i = hs(s)
            pltpu.sync_copy(
                x_hbm.at[h, pl.ds(sc_i * nrows, nrows)],
                idx_v.at[pl.ds((s % 2) * nrows, nrows)],
            )

        def fire_gather(c):
            s, half = divmod(c, 2)
            rows, sem = (rows_a, sem_a) if c % 2 == 0 else (rows_b, sem_b)
            return [
                pltpu.async_copy(
                    w_hbm.at[idx_v.at[(s % 2) * nrows + half * _C_ROWS + j]],
                    rows.at[pl.ds(j * _IDX_MINOR, _IDX_MINOR)],
                    sem,
                )
                for j in range(_C_ROWS)
            ]

        nchunks = 2 * _TPW
        inflight = [None] * nchunks
        fetch_idx(0)
        inflight[0] = fire_gather(0)
        for c in range(nchunks):
            s, half = divmod(c, 2)
            rows = rows_a if c % 2 == 0 else rows_b
            if c + 1 < nchunks:
                if (c + 1) % 2 == 0:
                    fetch_idx(s + 1)
                inflight[c + 1] = fire_gather(c + 1)
            for cp in inflight[c]:
                cp.wait()

            def row_body(r, c2, rows=rows):
                for j in range(_EMB_DIM // 16):
                    v = rows[r, pl.ds(j * 16, 16)]
                    t = v * inv_scale + zp
                    t = jnp.minimum(jnp.maximum(t, 0.0), _QMAX_F)
                    t = (t + _MAGIC) - _MAGIC
                    rows[r, pl.ds(j * 16, 16)] = (t - zp) * scale
                return c2

            lax.fori_loop(0, _C, row_body, 0)
            h, sc_i = hs(s)
            base = pl.multiple_of(sc_i * _SUPER + half * _C, _C)
            pltpu.sync_copy(rows, out_hbm.at[h, pl.ds(base, _C)])

    return k(xp3, weight, params)


def kernel(x, weight):
    w_rm, scale, zp = _tc_qparams(weight.T)
    w_lin = w_rm.reshape(2 * _SPLIT, _EMB_DIM)
    inv_scale = 1.0 / scale
    params = jnp.stack(
        [
            jnp.full((16,), inv_scale, jnp.float32),
            jnp.full((16,), zp, jnp.float32),
            jnp.full((16,), scale, jnp.float32),
            jnp.zeros((16,), jnp.float32),
        ]
    )
    xi = x.T.astype(jnp.int32)
    xm = jnp.where(xi < _SPLIT, 2 * xi, 2 * (xi - _SPLIT) + 1)
    xp3 = xm.reshape(_HIST, _BATCH // _IDX_MINOR, _IDX_MINOR)
    out3 = _sc_gather_quant(xp3, w_lin, params)
    return out3.transpose(1, 0, 2)


# speedup vs baseline: 1.5444x; 1.0485x over previous
import functools

import jax
import jax.numpy as jnp
from jax import lax
from jax.experimental import pallas as pl
from jax.experimental.pallas import tpu as pltpu
from jax.experimental.pallas import tpu_sc as plsc

_NUM_EMB = 1000000
_EMB_DIM = 64
_BATCH = 16384
_HIST = 20
_QMAX_F = 65535.0
_EPS = 0.0001 / 65535
_MAGIC = 12582912.0

_NW = 32
_C = 512
_IDX_MINOR = 128
_C_ROWS = _C // _IDX_MINOR
_SUPER = 1024
_BCHUNKS = _BATCH // _SUPER
_NTILE = _HIST * _BCHUNKS
_TPW = _NTILE // _NW

_SPLIT = 524288
_BW = 8192
_GRID_A = _SPLIT // _BW


def _qparams_body(wa_ref, wb_ref, out_ref, scale_ref, zp_ref, acc_ref):
    i = pl.program_id(0)
    a = wa_ref[...]
    b = wb_ref[...]
    colb = (
        _SPLIT + i * _BW
        + lax.broadcasted_iota(jnp.int32, (_EMB_DIM, _BW), 1)
    )
    validb = colb < _NUM_EMB
    out_ref[:, 0:_EMB_DIM] = a.T
    out_ref[:, _EMB_DIM:2 * _EMB_DIM] = b.T
    bmin = jnp.minimum(jnp.min(a), jnp.min(jnp.where(validb, b, jnp.inf)))
    bmax = jnp.maximum(jnp.max(a), jnp.max(jnp.where(validb, b, -jnp.inf)))

    @pl.when(i == 0)
    def _init():
        acc_ref[0] = bmin
        acc_ref[1] = bmax

    @pl.when(i > 0)
    def _acc():
        acc_ref[0] = jnp.minimum(acc_ref[0], bmin)
        acc_ref[1] = jnp.maximum(acc_ref[1], bmax)

    @pl.when(i == pl.num_programs(0) - 1)
    def _finish():
        mn = jnp.minimum(acc_ref[0], 0.0)
        mx = jnp.maximum(acc_ref[1], 0.0)
        sc = jnp.maximum((mx - mn) / _QMAX_F, jnp.float32(_EPS))
        zp = jnp.clip(-jnp.round(mn / sc), 0.0, _QMAX_F)
        scale_ref[0, 0] = sc
        zp_ref[0, 0] = zp


def _tc_qparams(wt):
    w_rm, scale, zp = pl.pallas_call(
        _qparams_body,
        grid=(_GRID_A,),
        in_specs=[
            pl.BlockSpec((_EMB_DIM, _BW), lambda i: (0, i)),
            pl.BlockSpec(
                (_EMB_DIM, _BW),
                lambda i: (0, jnp.minimum(i + _GRID_A, _NUM_EMB // _BW)),
            ),
        ],
        out_specs=[
            pl.BlockSpec((_BW, 2 * _EMB_DIM), lambda i: (i, 0)),
            pl.BlockSpec(memory_space=pltpu.SMEM),
            pl.BlockSpec(memory_space=pltpu.SMEM),
        ],
        out_shape=[
            jax.ShapeDtypeStruct((_SPLIT, 2 * _EMB_DIM), jnp.float32),
            jax.ShapeDtypeStruct((1, 1), jnp.float32),
            jax.ShapeDtypeStruct((1, 1), jnp.float32),
        ],
        scratch_shapes=[pltpu.SMEM((2,), jnp.float32)],
    )(wt, wt)
    return w_rm, scale[0, 0], zp[0, 0]


def _sc_gather_quant(xp3, weight, params):
    mesh = plsc.VectorSubcoreMesh(core_axis_name="c", subcore_axis_name="s")

    @functools.partial(
        pl.kernel,
        mesh=mesh,
        compiler_params=pltpu.CompilerParams(
            use_tc_tiling_on_sc=False, needs_layout_passes=False
        ),
        out_type=jax.ShapeDtypeStruct((_HIST, _BATCH, _EMB_DIM), jnp.float32),
        scratch_types=[
            pltpu.VMEM((2 * _SUPER // _IDX_MINOR, _IDX_MINOR), jnp.int32),
            pltpu.VMEM((_C, _EMB_DIM), jnp.float32),
            pltpu.VMEM((_C, _EMB_DIM), jnp.float32),
            pltpu.VMEM((4, 16), jnp.float32),
            pltpu.SemaphoreType.DMA,
            pltpu.SemaphoreType.DMA,
        ],
    )
    def k(x_hbm, w_hbm, p_hbm, out_hbm, idx_v, rows_a, rows_b, p_v, sem_a, sem_b):
        wid = lax.axis_index("s") * 2 + lax.axis_index("c")
        pltpu.sync_copy(p_hbm, p_v)
        inv_scale = p_v[0, :]
        zp = p_v[1, :]
        scale = p_v[2, :]
        nrows = _SUPER // _IDX_MINOR

        def hs(s):
            st = wid * _TPW + s
            return st // _BCHUNKS, st % _BCHUNKS

        def fetch_idx(s):
            h, sc_i = hs(s)
            pltpu.sync_copy(
                x_hbm.at[h, pl.ds(sc_i * nrows, nrows)],
                idx_v.at[pl.ds((s % 2) * nrows, nrows)],
            )

        def fire_gather(c):
            s, half = divmod(c, 2)
            rows, sem = (rows_a, sem_a) if c % 2 == 0 else (rows_b, sem_b)
            return [
                pltpu.async_copy(
                    w_hbm.at[idx_v.at[(s % 2) * nrows + half * _C_ROWS + j]],
                    rows.at[pl.ds(j * _IDX_MINOR, _IDX_MINOR)],
                    sem,
                )
                for j in range(_C_ROWS)
            ]

        nchunks = 2 * _TPW
        inflight = [None] * nchunks
        fetch_idx(0)
        inflight[0] = fire_gather(0)
        for c in range(nchunks):
            s, half = divmod(c, 2)
            rows = rows_a if c % 2 == 0 else rows_b
            if c + 1 < nchunks:
                if (c + 1) % 2 == 0:
                    fetch_idx(s + 1)
                inflight[c + 1] = fire_gather(c + 1)
            for cp in inflight[c]:
                cp.wait()

            def row_body(r, c2, rows=rows):
                for j in range(_EMB_DIM // 16):
                    v = rows[r, pl.ds(j * 16, 16)]
                    t = v * inv_scale + zp
                    t = jnp.minimum(jnp.maximum(t, 0.0), _QMAX_F)
                    t = (t + _MAGIC) - _MAGIC
                    rows[r, pl.ds(j * 16, 16)] = (t - zp) * scale
                return c2

            lax.fori_loop(0, _C, row_body, 0)
            h, sc_i = hs(s)
            base = pl.multiple_of(sc_i * _SUPER + half * _C, _C)
            pltpu.sync_copy(rows, out_hbm.at[h, pl.ds(base, _C)])

    return k(xp3, weight, params)


def kernel(x, weight):
    w_rm, scale, zp = _tc_qparams(weight.T)
    w_lin = w_rm.reshape(2 * _SPLIT, _EMB_DIM)
    inv_scale = 1.0 / scale
    params = jnp.stack(
        [
            jnp.full((16,), inv_scale, jnp.float32),
            jnp.full((16,), zp, jnp.float32),
            jnp.full((16,), scale, jnp.float32),
            jnp.zeros((16,), jnp.float32),
        ]
    )
    xi = x.T.astype(jnp.int32)
    xm = jnp.where(xi < _SPLIT, 2 * xi, 2 * (xi - _SPLIT) + 1)
    xp3 = xm.reshape(_HIST, _BATCH // _IDX_MINOR, _IDX_MINOR)
    out3 = _sc_gather_quant(xp3, w_lin, params)
    return out3.transpose(1, 0, 2)
